# Initial kernel scaffold; baseline (speedup 1.0000x reference)
import jax
import jax.numpy as jnp
from jax.experimental import pallas as pl


def kernel(input_ids, position_ids, word_embeddings, position_embeddings):
    raise NotImplementedError("write your pallas kernel here")


# speedup vs baseline: 1.1210x; 1.1210x over previous
import functools

import jax
import jax.numpy as jnp
from jax import lax
from jax.experimental import pallas as pl
from jax.experimental.pallas import tpu as pltpu
from jax.experimental.pallas import tpu_sc as plsc

NC = 2
NS = 16
NW = NC * NS

BATCH = 1024
SEQ = 200
D = 64
N = BATCH * SEQ
PER_W = N // NW
G = 128
NG = PER_W // G
LANES = 16
QUARTERS = D // LANES

_MESH = plsc.VectorSubcoreMesh(
    core_axis_name="c", subcore_axis_name="s", num_cores=NC, num_subcores=NS
)


@functools.partial(
    pl.kernel,
    out_type=jax.ShapeDtypeStruct((N, D), jnp.float32),
    mesh=_MESH,
    compiler_params=pltpu.CompilerParams(use_tc_tiling_on_sc=False),
    scratch_types=[
        pltpu.VMEM((PER_W,), jnp.int32),
        pltpu.VMEM((PER_W,), jnp.int32),
        pltpu.VMEM((G, D), jnp.float32),
        pltpu.VMEM((G, D), jnp.float32),
        pltpu.SemaphoreType.DMA,
        pltpu.SemaphoreType.DMA,
    ],
)
def _embed_sc(iid_hbm, pid_hbm, word_hbm, pos_hbm, out_hbm,
              idx_v, pidx_v, wrows_v, prows_v, sem_w, sem_p):
    wid = lax.axis_index("s") * NC + lax.axis_index("c")
    base = wid * PER_W

    pltpu.sync_copy(iid_hbm.at[pl.ds(base, PER_W)], idx_v)
    pltpu.sync_copy(pid_hbm.at[pl.ds(base, PER_W)], pidx_v)

    def group_body(g, carry):
        cp_w = pltpu.async_copy(word_hbm.at[idx_v.at[pl.ds(g * G, G)]], wrows_v, sem_w)
        cp_p = pltpu.async_copy(pos_hbm.at[pidx_v.at[pl.ds(g * G, G)]], prows_v, sem_p)
        cp_w.wait()
        cp_p.wait()

        def add_body(r, c):
            for q in range(QUARTERS):
                sl = pl.ds(q * LANES, LANES)
                wrows_v[r, sl] = wrows_v[r, sl] + prows_v[r, sl]
            return c

        lax.fori_loop(0, G, add_body, 0, unroll=2)
        pltpu.sync_copy(wrows_v, out_hbm.at[pl.ds(base + g * G, G)])
        return carry

    lax.fori_loop(0, NG, group_body, 0)


def kernel(input_ids, position_ids, word_embeddings, position_embeddings):
    iid = input_ids.reshape(N)
    pid = position_ids.reshape(N)
    out = _embed_sc(iid, pid, word_embeddings, position_embeddings)
    return out.reshape(BATCH, SEQ, D)


# speedup vs baseline: 1.2236x; 1.0915x over previous
import functools

import jax
import jax.numpy as jnp
from jax import lax
from jax.experimental import pallas as pl
from jax.experimental.pallas import tpu as pltpu
from jax.experimental.pallas import tpu_sc as plsc

NC = 2
NS = 16
NW = NC * NS

BATCH = 1024
SEQ = 200
D = 64
N = BATCH * SEQ
PER_W = N // NW
G = 128
NG = PER_W // G
NBUF = 5
LANES = 16
QUARTERS = D // LANES

_MESH = plsc.VectorSubcoreMesh(
    core_axis_name="c", subcore_axis_name="s", num_cores=NC, num_subcores=NS
)


@functools.partial(
    pl.kernel,
    out_type=jax.ShapeDtypeStruct((N, D), jnp.float32),
    mesh=_MESH,
    compiler_params=pltpu.CompilerParams(use_tc_tiling_on_sc=False),
    scratch_types=[
        pltpu.VMEM((PER_W,), jnp.int32),
        pltpu.VMEM((PER_W,), jnp.int32),
        pltpu.VMEM((NBUF, G, D), jnp.float32),
        pltpu.VMEM((NBUF, G, D), jnp.float32),
        pltpu.SemaphoreType.DMA((NBUF,)),
        pltpu.SemaphoreType.DMA((NBUF,)),
        pltpu.SemaphoreType.DMA((NBUF,)),
    ],
)
def _embed_sc(iid_hbm, pid_hbm, word_hbm, pos_hbm, out_hbm,
              idx_v, pidx_v, wr_v, pr_v, sem_w, sem_p, sem_o):
    wid = lax.axis_index("s") * NC + lax.axis_index("c")
    base = wid * PER_W

    pltpu.sync_copy(iid_hbm.at[pl.ds(base, PER_W)], idx_v)
    pltpu.sync_copy(pid_hbm.at[pl.ds(base, PER_W)], pidx_v)

    def fire_gathers(g, b):
        isl = pl.ds(g * G, G)
        pltpu.async_copy(word_hbm.at[idx_v.at[isl]], wr_v.at[b], sem_w.at[b])
        pltpu.async_copy(pos_hbm.at[pidx_v.at[isl]], pr_v.at[b], sem_p.at[b])

    def wait_bytes(dst, sem):
        pltpu.make_async_copy(word_hbm.at[pl.ds(0, G)], dst, sem).wait()

    for b in range(NBUF - 1):
        fire_gathers(b, b)

    def outer(j, carry):
        for b in range(NBUF):
            g = j * NBUF + b
            ga = g + NBUF - 1
            sa = (b + NBUF - 1) % NBUF

            @pl.when(jnp.logical_and(ga >= NBUF, ga < NG))
            def _():
                wait_bytes(wr_v.at[sa], sem_o.at[sa])

            @pl.when(ga < NG)
            def _():
                fire_gathers(ga, sa)

            wait_bytes(wr_v.at[b], sem_w.at[b])
            wait_bytes(pr_v.at[b], sem_p.at[b])

            def add_body(r, c):
                for q in range(QUARTERS):
                    sl = pl.ds(q * LANES, LANES)
                    wr_v[b, r, sl] = wr_v[b, r, sl] + pr_v[b, r, sl]
                return c

            lax.fori_loop(0, G, add_body, 0, unroll=4)
            pltpu.async_copy(
                wr_v.at[b], out_hbm.at[pl.ds(base + g * G, G)], sem_o.at[b]
            )
        return carry

    lax.fori_loop(0, NG // NBUF, outer, 0)

    for b in range(NBUF):
        wait_bytes(wr_v.at[b], sem_o.at[b])


def kernel(input_ids, position_ids, word_embeddings, position_embeddings):
    iid = input_ids.reshape(N)
    pid = position_ids.reshape(N)
    out = _embed_sc(iid, pid, word_embeddings, position_embeddings)
    return out.reshape(BATCH, SEQ, D)


# speedup vs baseline: 1.2885x; 1.0531x over previous
import functools

import jax
import jax.numpy as jnp
from jax import lax
from jax.experimental import pallas as pl
from jax.experimental.pallas import tpu as pltpu
from jax.experimental.pallas import tpu_sc as plsc

NC = 2
NS = 16
NW = NC * NS

BATCH = 1024
SEQ = 200
D = 64
MAX_POS = 201
N = BATCH * SEQ
PER_W = N // NW
G = 128
NG = PER_W // G
NBUF = 5
LANES = 16
QUARTERS = D // LANES

_MESH = plsc.VectorSubcoreMesh(
    core_axis_name="c", subcore_axis_name="s", num_cores=NC, num_subcores=NS
)


@functools.partial(
    pl.kernel,
    out_type=jax.ShapeDtypeStruct((N, D), jnp.float32),
    mesh=_MESH,
    compiler_params=pltpu.CompilerParams(use_tc_tiling_on_sc=False),
    scratch_types=[
        pltpu.VMEM((PER_W,), jnp.int32),
        pltpu.VMEM((PER_W,), jnp.int32),
        pltpu.VMEM((NBUF, G, D), jnp.float32),
        pltpu.VMEM((NBUF, G, D), jnp.float32),
        pltpu.VMEM_SHARED((MAX_POS, D), jnp.float32),
        pltpu.SemaphoreType.DMA((NBUF,)),
        pltpu.SemaphoreType.DMA((NBUF,)),
        pltpu.SemaphoreType.DMA((NBUF,)),
    ],
)
def _embed_sc(iid_hbm, pid_hbm, word_hbm, pos_hbm, out_hbm,
              idx_v, pidx_v, wr_v, pr_v, pos_sh, sem_w, sem_p, sem_o):
    sid = lax.axis_index("s")
    wid = sid * NC + lax.axis_index("c")
    base = wid * PER_W

    @pl.when(sid == 0)
    def _():
        pltpu.sync_copy(pos_hbm, pos_sh)

    pltpu.sync_copy(iid_hbm.at[pl.ds(base, PER_W)], idx_v)
    pltpu.sync_copy(pid_hbm.at[pl.ds(base, PER_W)], pidx_v)
    plsc.subcore_barrier()

    def fire_gathers(g, b):
        isl = pl.ds(g * G, G)
        pltpu.async_copy(word_hbm.at[idx_v.at[isl]], wr_v.at[b], sem_w.at[b])
        pltpu.async_copy(pos_sh.at[pidx_v.at[isl]], pr_v.at[b], sem_p.at[b])

    def wait_bytes(dst, sem):
        pltpu.make_async_copy(word_hbm.at[pl.ds(0, G)], dst, sem).wait()

    for b in range(NBUF - 1):
        fire_gathers(b, b)

    def outer(j, carry):
        for b in range(NBUF):
            g = j * NBUF + b
            ga = g + NBUF - 1
            sa = (b + NBUF - 1) % NBUF

            @pl.when(jnp.logical_and(ga >= NBUF, ga < NG))
            def _():
                wait_bytes(wr_v.at[sa], sem_o.at[sa])

            @pl.when(ga < NG)
            def _():
                fire_gathers(ga, sa)

            wait_bytes(wr_v.at[b], sem_w.at[b])
            wait_bytes(pr_v.at[b], sem_p.at[b])

            def add_body(r, c):
                for q in range(QUARTERS):
                    sl = pl.ds(q * LANES, LANES)
                    wr_v[b, r, sl] = wr_v[b, r, sl] + pr_v[b, r, sl]
                return c

            lax.fori_loop(0, G, add_body, 0, unroll=4)
            pltpu.async_copy(
                wr_v.at[b], out_hbm.at[pl.ds(base + g * G, G)], sem_o.at[b]
            )
        return carry

    lax.fori_loop(0, NG // NBUF, outer, 0)

    for b in range(NBUF):
        wait_bytes(wr_v.at[b], sem_o.at[b])


def kernel(input_ids, position_ids, word_embeddings, position_embeddings):
    iid = input_ids.reshape(N)
    pid = position_ids.reshape(N)
    out = _embed_sc(iid, pid, word_embeddings, position_embeddings)
    return out.reshape(BATCH, SEQ, D)
